# baseline (speedup 1.0000x reference)
import functools

import jax
import jax.numpy as jnp
from jax import lax
from jax.experimental import pallas as pl
from jax.experimental.pallas import tpu as pltpu
from jax.experimental.pallas import tpu_sc as plsc

B, D, T = 8, 512, 2048
NC, NS, L = 2, 16, 16
NW = NC * NS
DCH = D // NW
TCHUNK = 256
NTB = 8

_mesh = plsc.VectorSubcoreMesh(core_axis_name="c", subcore_axis_name="s")


@functools.partial(
    pl.kernel,
    mesh=_mesh,
    out_type=jax.ShapeDtypeStruct((B, D), jnp.float32),
    scratch_types=[
        pltpu.VMEM((2, DCH, T), jnp.float32),
        pltpu.VMEM((B * DCH,), jnp.float32),
        pltpu.VMEM((2 * L,), jnp.int32),
        pltpu.SemaphoreType.DMA((2,)),
        pltpu.SemaphoreType.DMA,
    ],
)
def _pool_kernel(x_hbm, len_hbm, out_hbm, buf, outv, lenv, sems, semo):
    wid = lax.axis_index("s") * NC + lax.axis_index("c")
    d0 = wid * DCH
    pltpu.async_copy(
        x_hbm.at[0, pl.ds(d0, DCH), pl.ds(0, TCHUNK)],
        buf.at[0, :, pl.ds(0, TCHUNK)],
        sems.at[0],
    )
    pltpu.sync_copy(len_hbm, lenv.at[pl.ds(0, B)])
    lane = jnp.arange(L, dtype=jnp.int32)
    neg_inf = jnp.full((L,), -jnp.inf, dtype=jnp.float32)

    def nch_of(b):
        n = lenv[pl.ds(b, L)][0]
        return n, (n + (TCHUNK - 1)) // TCHUNK

    def fire(b, n, nch, first=0):
        pb = b % 2

        def c_body(c, carry):
            pltpu.async_copy(
                x_hbm.at[b, pl.ds(d0, DCH), pl.ds(c * TCHUNK, TCHUNK)],
                buf.at[pb, :, pl.ds(c * TCHUNK, TCHUNK)],
                sems.at[pb],
            )
            return carry

        lax.fori_loop(first, nch, c_body, 0)

    def drain(b, nch):
        pb = b % 2

        def c_body(c, carry):
            pltpu.make_async_copy(
                x_hbm.at[b, pl.ds(d0, DCH), pl.ds(c * TCHUNK, TCHUNK)],
                buf.at[pb, :, pl.ds(c * TCHUNK, TCHUNK)],
                sems.at[pb],
            ).wait()
            return carry

        lax.fori_loop(0, nch, c_body, 0)

    def compute(b, n):
        pb = b % 2
        nu = n // (NTB * L)
        tb = jnp.minimum(nu * (NTB * L), T - NTB * L)
        masks = [(tb + (i * L) + lane) < n for i in range(NTB)]

        def row_body(rr, ovec):
            rows = [4 * rr + j for j in range(4)]

            def k_body(k, accs):
                accs = list(accs)
                base = k * (NTB * L)
                for i in range(NTB):
                    for j in range(4):
                        x = buf[pb, rows[j], pl.ds(base + i * L, L)]
                        c = 2 * j + (i % 2)
                        accs[c] = jnp.maximum(accs[c], x)
                return tuple(accs)

            accs = lax.fori_loop(0, nu, k_body, (neg_inf,) * 8)
            accs = list(accs)
            for i in range(NTB):
                for j in range(4):
                    x = buf[pb, rows[j], pl.ds(tb + i * L, L)]
                    x = jnp.where(masks[i], x, neg_inf)
                    c = 2 * j + (i % 2)
                    accs[c] = jnp.maximum(accs[c], x)
            folded = [jnp.maximum(accs[2 * j], accs[2 * j + 1]) for j in range(4)]
            for s in (8, 4, 2, 1):
                folded = [
                    jnp.maximum(a, jnp.take_along_axis(a, lane ^ s, axis=0))
                    for a in folded
                ]
            for j in range(4):
                ovec = jnp.where(lane == rows[j], folded[j], ovec)
            return ovec

        ovec = lax.fori_loop(0, DCH // 4, row_body, neg_inf)
        outv[pl.ds(b * DCH, DCH)] = ovec
        pltpu.async_copy(
            outv.at[pl.ds(b * DCH, DCH)],
            out_hbm.at[b, pl.ds(d0, DCH)],
            semo,
        )

    n0, nch0 = nch_of(0)
    fire(0, n0, nch0, first=1)

    def batch_body(b, state):
        n, nch = state
        nxt = lax.cond(
            b + 1 < B, lambda: nch_of(b + 1), lambda: (n, jnp.int32(0))
        )
        fire(b + 1, *nxt)
        drain(b, nch)
        compute(b, n)
        return nxt

    lax.fori_loop(0, B, batch_body, (n0, nch0))

    def out_drain(b, carry):
        pltpu.make_async_copy(
            outv.at[pl.ds(b * DCH, DCH)],
            out_hbm.at[b, pl.ds(d0, DCH)],
            semo,
        ).wait()
        return carry

    lax.fori_loop(0, B, out_drain, 0)


def kernel(x0, x1, x2):
    del x1
    return _pool_kernel(x0, x2.astype(jnp.int32))
